# baseline (speedup 1.0000x reference)
import jax
import jax.numpy as jnp
from jax.experimental import pallas as pl

_LANES = 128
_GROUP = 16
_GPL = _LANES // _GROUP


def _mask_body(x_ref, o_ref):
    x = x_ref[...]
    blk, flatdim = x.shape
    sq = x * x

    li = jax.lax.broadcasted_iota(jnp.int32, (_LANES, _LANES), 0)
    co = jax.lax.broadcasted_iota(jnp.int32, (_LANES, _LANES), 1)
    gmat = (li // _GROUP == co // _GROUP).astype(jnp.float32)
    parts = []
    for r in range(flatdim // _LANES):
        parts.append(
            jax.lax.dot_general(
                sq[:, r * _LANES:(r + 1) * _LANES], gmat,
                (((1,), (0,)), ((), ())),
                preferred_element_type=jnp.float32,
                precision=jax.lax.Precision.HIGHEST,
            )
        )
    sg = jnp.concatenate(parts, axis=1)

    m = jnp.max(sg, axis=1, keepdims=True)

    flat = jax.lax.broadcasted_iota(jnp.int32, (1, flatdim), 1) // _GROUP

    big = jnp.int32(1 << 30)
    wh = jnp.where(sg == m, flat, big)
    idx = jnp.min(wh, axis=1, keepdims=True)

    keep = flat == idx
    o_ref[...] = jnp.where(keep, x, jnp.float32(0.0))


def kernel(inputs):
    b, c, d = inputs.shape
    flat = c * d
    blk = 32
    x = inputs.reshape(b, flat)
    return pl.pallas_call(
        _mask_body,
        grid=(b // blk,),
        in_specs=[pl.BlockSpec((blk, flat), lambda i: (i, 0))],
        out_specs=pl.BlockSpec((blk, flat), lambda i: (i, 0)),
        out_shape=jax.ShapeDtypeStruct((b, flat), jnp.float32),
    )(x)


# speedup vs baseline: 2.4735x; 2.4735x over previous
import jax
import jax.numpy as jnp
from jax.experimental import pallas as pl

_LANES = 128
_GROUP = 16
_GPL = _LANES // _GROUP


def _mask_body(x_ref, o_ref):
    x = x_ref[...]
    blk, flatdim = x.shape
    sq = x * x

    hi = sq.astype(jnp.bfloat16)
    r1 = sq - hi.astype(jnp.float32)
    mid = r1.astype(jnp.bfloat16)
    r2 = r1 - mid.astype(jnp.float32)
    lo = r2.astype(jnp.bfloat16)

    li = jax.lax.broadcasted_iota(jnp.int32, (_LANES, _LANES), 0)
    co = jax.lax.broadcasted_iota(jnp.int32, (_LANES, _LANES), 1)
    gmat = (li // _GROUP == co // _GROUP).astype(jnp.bfloat16)
    gmat3 = jnp.concatenate([gmat, gmat, gmat], axis=0)
    parts = []
    for r in range(flatdim // _LANES):
        sl = slice(r * _LANES, (r + 1) * _LANES)
        lhs = jnp.concatenate([hi[:, sl], mid[:, sl], lo[:, sl]], axis=1)
        parts.append(
            jax.lax.dot_general(
                lhs, gmat3, (((1,), (0,)), ((), ())),
                preferred_element_type=jnp.float32,
            )
        )
    sg = jnp.concatenate(parts, axis=1)

    m = jnp.max(sg, axis=1, keepdims=True)

    flat = jax.lax.broadcasted_iota(jnp.int32, (1, flatdim), 1) // _GROUP

    big = jnp.int32(1 << 30)
    wh = jnp.where(sg == m, flat, big)
    idx = jnp.min(wh, axis=1, keepdims=True)

    keep = flat == idx
    o_ref[...] = jnp.where(keep, x, jnp.float32(0.0))


def kernel(inputs):
    b, c, d = inputs.shape
    flat = c * d
    blk = 64
    x = inputs.reshape(b, flat)
    return pl.pallas_call(
        _mask_body,
        grid=(b // blk,),
        in_specs=[pl.BlockSpec((blk, flat), lambda i: (i, 0))],
        out_specs=pl.BlockSpec((blk, flat), lambda i: (i, 0)),
        out_shape=jax.ShapeDtypeStruct((b, flat), jnp.float32),
    )(x)


# speedup vs baseline: 2.7177x; 1.0987x over previous
import jax
import jax.numpy as jnp
from jax.experimental import pallas as pl

_LANES = 128
_GROUP = 16
_GPL = _LANES // _GROUP


def _mask_body(x_ref, o_ref):
    x = x_ref[...]
    blk, flatdim = x.shape
    sq = x * x

    hi = sq.astype(jnp.bfloat16)
    r1 = sq - hi.astype(jnp.float32)
    mid = r1.astype(jnp.bfloat16)
    r2 = r1 - mid.astype(jnp.float32)
    lo = r2.astype(jnp.bfloat16)

    li = jax.lax.broadcasted_iota(jnp.int32, (_LANES, _LANES), 0)
    co = jax.lax.broadcasted_iota(jnp.int32, (_LANES, _LANES), 1)
    gmat = (li // _GROUP == co // _GROUP).astype(jnp.bfloat16)
    gmat3 = jnp.concatenate([gmat, gmat, gmat], axis=0)
    parts = []
    for r in range(flatdim // _LANES):
        sl = slice(r * _LANES, (r + 1) * _LANES)
        lhs = jnp.concatenate([hi[:, sl], mid[:, sl], lo[:, sl]], axis=1)
        parts.append(
            jax.lax.dot_general(
                lhs, gmat3, (((1,), (0,)), ((), ())),
                preferred_element_type=jnp.float32,
            )
        )
    sg = jnp.concatenate(parts, axis=1)

    m = jnp.max(sg, axis=1, keepdims=True)

    flat = jax.lax.broadcasted_iota(jnp.int32, (1, flatdim), 1) // _GROUP

    big = jnp.int32(1 << 30)
    wh = jnp.where(sg == m, flat, big)
    idx = jnp.min(wh, axis=1, keepdims=True)

    keep = flat == idx
    o_ref[...] = jnp.where(keep, x, jnp.float32(0.0))


def kernel(inputs):
    b, c, d = inputs.shape
    flat = c * d
    blk = 128
    x = inputs.reshape(b, flat)
    return pl.pallas_call(
        _mask_body,
        grid=(b // blk,),
        in_specs=[pl.BlockSpec((blk, flat), lambda i: (i, 0))],
        out_specs=pl.BlockSpec((blk, flat), lambda i: (i, 0)),
        out_shape=jax.ShapeDtypeStruct((b, flat), jnp.float32),
    )(x)


# speedup vs baseline: 2.7199x; 1.0008x over previous
import jax
import jax.numpy as jnp
from jax.experimental import pallas as pl

_LANES = 128
_GROUP = 16
_GPL = _LANES // _GROUP


def _mask_body(x_ref, o_ref):
    x = x_ref[...]
    blk, flatdim = x.shape

    li = jax.lax.broadcasted_iota(jnp.int32, (_LANES, _LANES), 0)
    co = jax.lax.broadcasted_iota(jnp.int32, (_LANES, _LANES), 1)
    gmat = (li // _GROUP == co // _GROUP).astype(jnp.bfloat16)
    gmat3 = jnp.concatenate([gmat, gmat, gmat], axis=0)
    parts = []
    for r in range(flatdim // _LANES):
        sl = slice(r * _LANES, (r + 1) * _LANES)
        sqt = x[:, sl] * x[:, sl]
        hit = sqt.astype(jnp.bfloat16)
        r1t = sqt - hit.astype(jnp.float32)
        midt = r1t.astype(jnp.bfloat16)
        r2t = r1t - midt.astype(jnp.float32)
        lot = r2t.astype(jnp.bfloat16)
        lhs = jnp.concatenate([hit, midt, lot], axis=1)
        parts.append(
            jax.lax.dot_general(
                lhs, gmat3, (((1,), (0,)), ((), ())),
                preferred_element_type=jnp.float32,
            )
        )
    sg = jnp.concatenate(parts, axis=1)

    m = jnp.max(sg, axis=1, keepdims=True)

    flat = jax.lax.broadcasted_iota(jnp.int32, (1, flatdim), 1) // _GROUP

    big = jnp.int32(1 << 30)
    wh = jnp.where(sg == m, flat, big)
    idx = jnp.min(wh, axis=1, keepdims=True)

    keep = flat == idx
    o_ref[...] = jnp.where(keep, x, jnp.float32(0.0))


def kernel(inputs):
    b, c, d = inputs.shape
    flat = c * d
    blk = 128
    x = inputs.reshape(b, flat)
    return pl.pallas_call(
        _mask_body,
        grid=(b // blk,),
        in_specs=[pl.BlockSpec((blk, flat), lambda i: (i, 0))],
        out_specs=pl.BlockSpec((blk, flat), lambda i: (i, 0)),
        out_shape=jax.ShapeDtypeStruct((b, flat), jnp.float32),
    )(x)
